# baseline (speedup 1.0000x reference)
import dataclasses
import functools

import jax
import jax.numpy as jnp
from jax import lax
from jax.experimental import pallas as pl
from jax.experimental.pallas import tpu as pltpu
from jax.experimental.pallas import tpu_sc as plsc

N = 10000
H = 256
S = 8
NC = 2
NS = 16
NW = NC * NS
TPW = 320
NPAD = NW * TPW
C = 16
CHUNKS = TPW // C
BLK = 1000
GRID = N // BLK


def _sc_gather_sums(h1, h2, idx0, idx1):
    mesh = plsc.VectorSubcoreMesh(core_axis_name="c", subcore_axis_name="s")
    cp = pltpu.CompilerParams()
    if "needs_layout_passes" in pltpu.CompilerParams.__dataclass_fields__:
        cp = dataclasses.replace(cp, needs_layout_passes=False)

    @functools.partial(
        pl.kernel,
        compiler_params=cp,
        out_type=(
            jax.ShapeDtypeStruct((NPAD, H // 2), jnp.int32),
            jax.ShapeDtypeStruct((NPAD, H // 2), jnp.int32),
        ),
        mesh=mesh,
        scratch_types=[
            pltpu.VMEM((TPW * S,), jnp.int32),
            pltpu.VMEM((C * S, H // 2), jnp.int32),
            pltpu.VMEM((C * S, H // 2), jnp.int32),
            pltpu.VMEM((C, H // 2), jnp.int32),
            pltpu.VMEM((C, H // 2), jnp.int32),
            pltpu.SemaphoreType.DMA,
            pltpu.SemaphoreType.DMA,
            pltpu.SemaphoreType.DMA,
            pltpu.SemaphoreType.DMA,
        ],
    )
    def sc_kernel(h1_hbm, h2_hbm, i0_hbm, i1_hbm, o0_hbm, o1_hbm,
                  idx_v, rows0, rows1, acc0, acc1, gs0, gs1, ws0, ws1):
        wid = lax.axis_index("s") * NC + lax.axis_index("c")
        tbase = wid * TPW
        ibase = tbase * S
        rows = (rows0, rows1)
        accs = (acc0, acc1)
        gsem = (gs0, gs1)
        wsem = (ws0, ws1)

        for t_hbm, i_hbm, o_hbm in ((h1_hbm, i0_hbm, o0_hbm),
                                    (h2_hbm, i1_hbm, o1_hbm)):
            pltpu.sync_copy(i_hbm.at[pl.ds(ibase, TPW * S)], idx_v)

            def g_start(cc, b, t_hbm=t_hbm):
                pltpu.make_async_copy(
                    t_hbm.at[idx_v.at[pl.ds(cc * (C * S), C * S)]],
                    rows[b], gsem[b]).start()

            def g_wait(b, t_hbm=t_hbm):
                pltpu.make_async_copy(
                    t_hbm.at[idx_v.at[pl.ds(0, C * S)]],
                    rows[b], gsem[b]).wait()

            def w_start(cc, b, o_hbm=o_hbm):
                pltpu.make_async_copy(
                    accs[b], o_hbm.at[pl.ds(tbase + cc * C, C)],
                    wsem[b]).start()

            def w_wait(b, o_hbm=o_hbm):
                pltpu.make_async_copy(
                    accs[b], o_hbm.at[pl.ds(tbase, C)], wsem[b]).wait()

            g_start(0, 0)

            @pl.loop(0, CHUNKS, step=2)
            def _(c, g_start=g_start, g_wait=g_wait,
                  w_start=w_start, w_wait=w_wait):
                for b in (0, 1):
                    cc = c + b

                    @pl.when(cc + 1 < CHUNKS)
                    def _(cc=cc, b=b):
                        g_start(cc + 1, 1 - b)

                    g_wait(b)

                    @pl.when(cc >= 2)
                    def _(b=b):
                        w_wait(b)

                    rb, ab = rows[b], accs[b]

                    @pl.loop(0, C)
                    def _(t, rb=rb, ab=ab):
                        r = t * S

                        def bf(x):
                            return plsc.bitcast(x, jnp.bfloat16)

                        for j in range(H // 32):
                            sl = pl.ds(j * 16, 16)
                            v01 = bf(rb[r, sl]) + bf(rb[r + 1, sl])
                            v23 = bf(rb[r + 2, sl]) + bf(rb[r + 3, sl])
                            v45 = bf(rb[r + 4, sl]) + bf(rb[r + 5, sl])
                            v67 = bf(rb[r + 6, sl]) + bf(rb[r + 7, sl])
                            ab[t, sl] = plsc.bitcast(
                                (v01 + v23) + (v45 + v67), jnp.int32)

                    w_start(cc, b)

            w_wait(0)
            w_wait(1)

    return sc_kernel(h1, h2, idx0, idx1)


def _tc_colsums(s0, s1, fc_wt, fc_b):
    def body(x0_ref, x1_ref, w_ref, b_ref, out_ref):
        @pl.when(pl.program_id(0) == 0)
        def _():
            out_ref[...] = jnp.zeros_like(out_ref)

        x0 = x0_ref[...].astype(jnp.float32)
        x1 = x1_ref[...].astype(jnp.float32)
        t0 = jnp.tanh(jnp.dot(x0, w_ref[...],
                              preferred_element_type=jnp.float32) + b_ref[...])
        t1 = jnp.tanh(jnp.dot(x1, w_ref[...],
                              preferred_element_type=jnp.float32) + b_ref[...])
        out_ref[0:1, :] += jnp.sum(t0, axis=0, keepdims=True)
        out_ref[1:2, :] += jnp.sum(t1, axis=0, keepdims=True)

    return pl.pallas_call(
        body,
        grid=(GRID,),
        in_specs=[
            pl.BlockSpec((BLK, H), lambda i: (i, 0)),
            pl.BlockSpec((BLK, H), lambda i: (i, 0)),
            pl.BlockSpec((H, H), lambda i: (0, 0)),
            pl.BlockSpec((1, H), lambda i: (0, 0)),
        ],
        out_specs=pl.BlockSpec((8, H), lambda i: (0, 0)),
        out_shape=jax.ShapeDtypeStruct((8, H), jnp.float32),
    )(s0, s1, fc_wt, fc_b)


def _tc_combine(cs, att, s0, s1, pred_wt, pred_b):
    def body(cs_ref, att_ref, x0_ref, x1_ref, w_ref, b_ref, out_ref):
        a = att_ref[0, :]
        v0 = jnp.sum(cs_ref[0, :] * a) * (1.0 / N)
        v1 = jnp.sum(cs_ref[1, :] * a) * (1.0 / N)
        m = jnp.maximum(v0, v1)
        e0 = jnp.exp(v0 - m)
        e1 = jnp.exp(v1 - m)
        inv = 1.0 / (e0 + e1)
        b0 = e0 * inv
        b1 = e1 * inv
        z = (x0_ref[...].astype(jnp.float32) * b0
             + x1_ref[...].astype(jnp.float32) * b1)
        out_ref[...] = jnp.tanh(
            jnp.dot(z, w_ref[...], preferred_element_type=jnp.float32)
            + b_ref[...])

    return pl.pallas_call(
        body,
        grid=(GRID,),
        in_specs=[
            pl.BlockSpec((8, H), lambda i: (0, 0)),
            pl.BlockSpec((1, H), lambda i: (0, 0)),
            pl.BlockSpec((BLK, H), lambda i: (i, 0)),
            pl.BlockSpec((BLK, H), lambda i: (i, 0)),
            pl.BlockSpec((H, H), lambda i: (0, 0)),
            pl.BlockSpec((1, H), lambda i: (0, 0)),
        ],
        out_specs=pl.BlockSpec((BLK, H), lambda i: (i, 0)),
        out_shape=jax.ShapeDtypeStruct((N, H), jnp.float32),
    )(cs, att, s0, s1, pred_wt, pred_b)


def kernel(h0, h1, h2, nei_idx0, nei_idx1, fc_W, fc_b, att, pred_W, pred_b):
    del h0
    idx0 = nei_idx0.astype(jnp.int32).reshape(-1)
    idx1 = nei_idx1.astype(jnp.int32).reshape(-1)
    pad = NPAD * S - idx0.shape[0]
    idx0 = jnp.concatenate([idx0, jnp.zeros((pad,), jnp.int32)])
    idx1 = jnp.concatenate([idx1, jnp.zeros((pad,), jnp.int32)])

    def pack(h):
        hb = h.astype(jnp.bfloat16).reshape(N, H // 2, 2)
        return jax.lax.bitcast_convert_type(hb, jnp.int32)

    def unpack(s):
        sb = jax.lax.bitcast_convert_type(s, jnp.bfloat16)
        return sb.reshape(NPAD, H)

    s0i, s1i = _sc_gather_sums(pack(h1), pack(h2), idx0, idx1)
    s0, s1 = unpack(s0i), unpack(s1i)

    fc_wt = fc_W.T * (1.0 / S)
    pred_wt = pred_W.T * (1.0 / S)
    cs = _tc_colsums(s0, s1, fc_wt, fc_b.reshape(1, H))
    out = _tc_combine(cs, att.reshape(1, H), s0, s1,
                      pred_wt, pred_b.reshape(1, H))
    return out


# speedup vs baseline: 1.8109x; 1.8109x over previous
import functools

import jax
import jax.numpy as jnp
from jax import lax
from jax.experimental import pallas as pl
from jax.experimental.pallas import tpu as pltpu
from jax.experimental.pallas import tpu_sc as plsc

N = 10000
H = 256
S = 8
NC = 2
NS = 16
TPW0 = 448
TPW1 = 192
NPAD = NS * (TPW0 + TPW1)
BASE1 = NS * TPW0
C = 16
BLK = 1000
GRID = N // BLK


def _sc_gather_sums(h1, h2, idx0, idx1):
    mesh = plsc.VectorSubcoreMesh(core_axis_name="c", subcore_axis_name="s")

    @functools.partial(
        pl.kernel,
        out_type=(
            jax.ShapeDtypeStruct((NPAD, H), jnp.float32),
            jax.ShapeDtypeStruct((NPAD, H), jnp.float32),
        ),
        mesh=mesh,
        scratch_types=[
            pltpu.VMEM((TPW0 * S,), jnp.int32),
            pltpu.VMEM((C * S, H), jnp.float32),
            pltpu.VMEM((C * S, H), jnp.float32),
            pltpu.VMEM((C, H), jnp.float32),
            pltpu.VMEM((C, H), jnp.float32),
            pltpu.SemaphoreType.DMA,
            pltpu.SemaphoreType.DMA,
            pltpu.SemaphoreType.DMA,
            pltpu.SemaphoreType.DMA,
        ],
    )
    def sc_kernel(h1_hbm, h2_hbm, i0_hbm, i1_hbm, o0_hbm, o1_hbm,
                  idx_v, rows0, rows1, acc0, acc1, gs0, gs1, ws0, ws1):
        core = lax.axis_index("c")
        sid = lax.axis_index("s")
        rows = (rows0, rows1)
        accs = (acc0, acc1)
        gsem = (gs0, gs1)
        wsem = (ws0, ws1)

        for ci, tpw in ((0, TPW0), (1, TPW1)):
            chunks = tpw // C

            @pl.when(core == ci)
            def _(ci=ci, tpw=tpw, chunks=chunks):
                tbase = sid * tpw + (BASE1 if ci == 1 else 0)
                ibase = tbase * S

                for t_hbm, i_hbm, o_hbm in ((h1_hbm, i0_hbm, o0_hbm),
                                            (h2_hbm, i1_hbm, o1_hbm)):
                    pltpu.sync_copy(i_hbm.at[pl.ds(ibase, tpw * S)],
                                    idx_v.at[pl.ds(0, tpw * S)])

                    def g_start(cc, b, t_hbm=t_hbm):
                        pltpu.make_async_copy(
                            t_hbm.at[idx_v.at[pl.ds(cc * (C * S), C * S)]],
                            rows[b], gsem[b]).start()

                    def g_wait(b, t_hbm=t_hbm):
                        pltpu.make_async_copy(
                            t_hbm.at[idx_v.at[pl.ds(0, C * S)]],
                            rows[b], gsem[b]).wait()

                    def w_start(cc, b, o_hbm=o_hbm, tbase=tbase):
                        pltpu.make_async_copy(
                            accs[b], o_hbm.at[pl.ds(tbase + cc * C, C)],
                            wsem[b]).start()

                    def w_wait(b, o_hbm=o_hbm, tbase=tbase):
                        pltpu.make_async_copy(
                            accs[b], o_hbm.at[pl.ds(tbase, C)],
                            wsem[b]).wait()

                    g_start(0, 0)

                    @pl.loop(0, chunks, step=2)
                    def _(c, g_start=g_start, g_wait=g_wait,
                          w_start=w_start, w_wait=w_wait, chunks=chunks):
                        for b in (0, 1):
                            cc = c + b

                            @pl.when(cc + 1 < chunks)
                            def _(cc=cc, b=b):
                                g_start(cc + 1, 1 - b)

                            g_wait(b)

                            @pl.when(cc >= 2)
                            def _(b=b):
                                w_wait(b)

                            rb, ab = rows[b], accs[b]

                            @pl.loop(0, C)
                            def _(t, rb=rb, ab=ab):
                                r = t * S
                                for j in range(H // 16):
                                    sl = pl.ds(j * 16, 16)
                                    v01 = rb[r, sl] + rb[r + 1, sl]
                                    v23 = rb[r + 2, sl] + rb[r + 3, sl]
                                    v45 = rb[r + 4, sl] + rb[r + 5, sl]
                                    v67 = rb[r + 6, sl] + rb[r + 7, sl]
                                    ab[t, sl] = (v01 + v23) + (v45 + v67)

                            w_start(cc, b)

                    w_wait(0)
                    w_wait(1)

    return sc_kernel(h1, h2, idx0, idx1)


def _tc_colsums(s0, s1, fc_wt, fc_b):
    def body(x0_ref, x1_ref, w_ref, b_ref, out_ref):
        @pl.when(pl.program_id(0) == 0)
        def _():
            out_ref[...] = jnp.zeros_like(out_ref)

        t0 = jnp.tanh(jnp.dot(x0_ref[...], w_ref[...],
                              preferred_element_type=jnp.float32) + b_ref[...])
        t1 = jnp.tanh(jnp.dot(x1_ref[...], w_ref[...],
                              preferred_element_type=jnp.float32) + b_ref[...])
        out_ref[0:1, :] += jnp.sum(t0, axis=0, keepdims=True)
        out_ref[1:2, :] += jnp.sum(t1, axis=0, keepdims=True)

    return pl.pallas_call(
        body,
        grid=(GRID,),
        in_specs=[
            pl.BlockSpec((BLK, H), lambda i: (i, 0)),
            pl.BlockSpec((BLK, H), lambda i: (i, 0)),
            pl.BlockSpec((H, H), lambda i: (0, 0)),
            pl.BlockSpec((1, H), lambda i: (0, 0)),
        ],
        out_specs=pl.BlockSpec((8, H), lambda i: (0, 0)),
        out_shape=jax.ShapeDtypeStruct((8, H), jnp.float32),
    )(s0, s1, fc_wt, fc_b)


def _tc_combine(cs, att, s0, s1, pred_wt, pred_b):
    def body(cs_ref, att_ref, x0_ref, x1_ref, w_ref, b_ref, out_ref):
        a = att_ref[0, :]
        v0 = jnp.sum(cs_ref[0, :] * a) * (1.0 / N)
        v1 = jnp.sum(cs_ref[1, :] * a) * (1.0 / N)
        m = jnp.maximum(v0, v1)
        e0 = jnp.exp(v0 - m)
        e1 = jnp.exp(v1 - m)
        inv = 1.0 / (e0 + e1)
        b0 = e0 * inv
        b1 = e1 * inv
        z = x0_ref[...] * b0 + x1_ref[...] * b1
        out_ref[...] = jnp.tanh(
            jnp.dot(z, w_ref[...], preferred_element_type=jnp.float32)
            + b_ref[...])

    return pl.pallas_call(
        body,
        grid=(GRID,),
        in_specs=[
            pl.BlockSpec((8, H), lambda i: (0, 0)),
            pl.BlockSpec((1, H), lambda i: (0, 0)),
            pl.BlockSpec((BLK, H), lambda i: (i, 0)),
            pl.BlockSpec((BLK, H), lambda i: (i, 0)),
            pl.BlockSpec((H, H), lambda i: (0, 0)),
            pl.BlockSpec((1, H), lambda i: (0, 0)),
        ],
        out_specs=pl.BlockSpec((BLK, H), lambda i: (i, 0)),
        out_shape=jax.ShapeDtypeStruct((N, H), jnp.float32),
    )(cs, att, s0, s1, pred_wt, pred_b)


def kernel(h0, h1, h2, nei_idx0, nei_idx1, fc_W, fc_b, att, pred_W, pred_b):
    del h0
    idx0 = nei_idx0.astype(jnp.int32).reshape(-1)
    idx1 = nei_idx1.astype(jnp.int32).reshape(-1)
    pad = NPAD * S - idx0.shape[0]
    idx0 = jnp.concatenate([idx0, jnp.zeros((pad,), jnp.int32)])
    idx1 = jnp.concatenate([idx1, jnp.zeros((pad,), jnp.int32)])

    s0, s1 = _sc_gather_sums(h1, h2, idx0, idx1)

    fc_wt = fc_W.T * (1.0 / S)
    pred_wt = pred_W.T * (1.0 / S)
    cs = _tc_colsums(s0, s1, fc_wt, fc_b.reshape(1, H))
    out = _tc_combine(cs, att.reshape(1, H), s0, s1,
                      pred_wt, pred_b.reshape(1, H))
    return out
